# Initial kernel scaffold; baseline (speedup 1.0000x reference)
import jax
import jax.numpy as jnp
from jax.experimental import pallas as pl


def kernel(x, edge_index, drnl, batch, target_local, emb, W1, b1, W2, b2, Wm1, bm1, Wm2, bm2):
    raise NotImplementedError("write your pallas kernel here")


# speedup vs baseline: 8.0221x; 8.0221x over previous
import functools

import jax
import jax.numpy as jnp
from jax import lax
from jax.experimental import pallas as pl
from jax.experimental.pallas import tpu as pltpu
from jax.experimental.pallas import tpu_sc as plsc

N = 10000
E = 320000
D = 128
HID = 128
G = 64

NC, NS = 2, 16
NW = NC * NS
N_PAD = 10240
PAD_NODE = N_PAD - 1
CH = 128
CPW = 79
EPW = CPW * CH
E_PAD = NW * EPW
E_HALF = E_PAD // 2
RPW = N_PAD // NW
RPS = N_PAD // NS
GCH = 80
ZROWS = 64

_sc_mesh = plsc.VectorSubcoreMesh(
    core_axis_name="c", subcore_axis_name="s", num_cores=NC, num_subcores=NS)


def _mm_body(a_ref, w_ref, o_ref):
    o_ref[...] = jnp.dot(a_ref[...], w_ref[...],
                         preferred_element_type=jnp.float32)


def _matmul(a, w, br):
    m, k = a.shape
    n = w.shape[1]
    return pl.pallas_call(
        _mm_body,
        grid=(m // br,),
        in_specs=[pl.BlockSpec((br, k), lambda i: (i, 0)),
                  pl.BlockSpec((k, n), lambda i: (0, 0))],
        out_specs=pl.BlockSpec((br, n), lambda i: (i, 0)),
        out_shape=jax.ShapeDtypeStruct((m, n), jnp.float32),
    )(a, w)


def _deg_sc_body(dst_hbm, degp_hbm, didx, ones_v, zbuf, acc):
    c = lax.axis_index("c")
    s = lax.axis_index("s")

    def zb(j, carry):
        for t in range(HID // 16):
            zbuf[j, pl.ds(t * 16, 16)] = jnp.zeros((16,), jnp.float32)
        return carry
    lax.fori_loop(0, ZROWS, zb, 0)

    def ob(j, carry):
        for t in range(HID // 16):
            ones_v[j, pl.ds(t * 16, 16)] = jnp.ones((16,), jnp.float32)
        return carry
    lax.fori_loop(0, CH, ob, 0)

    def zc(r, carry):
        pltpu.sync_copy(zbuf, acc.at[pl.ds(s * RPS + r * ZROWS, ZROWS)])
        return carry
    lax.fori_loop(0, RPS // ZROWS, zc, 0)
    plsc.subcore_barrier()

    ebase = c * E_HALF + s * EPW

    def eb(i, carry):
        pltpu.sync_copy(dst_hbm.at[pl.ds(ebase + i * CH, CH)], didx)
        pltpu.sync_copy(ones_v, acc.at[didx], add=True)
        return carry
    lax.fori_loop(0, CPW, eb, 0)

    plsc.subcore_barrier()
    pltpu.sync_copy(acc.at[pl.ds(s * RPS, RPS)],
                    degp_hbm.at[pl.ds(c * N_PAD + s * RPS, RPS)])


_sc_deg = pl.kernel(
    _deg_sc_body,
    out_type=jax.ShapeDtypeStruct((2 * N_PAD, HID), jnp.float32),
    mesh=_sc_mesh,
    scratch_types=[
        pltpu.VMEM((CH,), jnp.int32),
        pltpu.VMEM((CH, HID), jnp.float32),
        pltpu.VMEM((ZROWS, HID), jnp.float32),
        pltpu.VMEM_SHARED((N_PAD, HID), jnp.float32),
    ],
)


def _sc1_body(drnl_hbm, embw_hbm, xw_hbm, h1pre_hbm, gidx, grow, xrow, gsem):
    c = lax.axis_index("c")
    s = lax.axis_index("s")
    wid = s * NC + c
    nbase = wid * RPW
    for k in range(RPW // GCH):
        pltpu.sync_copy(drnl_hbm.at[pl.ds(nbase + k * GCH, GCH)], gidx)
        pltpu.async_copy(embw_hbm.at[gidx], grow, gsem).wait()
        pltpu.sync_copy(xw_hbm.at[pl.ds(nbase + k * GCH, GCH)], xrow)

        def ab(j, carry):
            for t in range(HID // 16):
                grow[j, pl.ds(t * 16, 16)] = (
                    grow[j, pl.ds(t * 16, 16)] + xrow[j, pl.ds(t * 16, 16)])
            return carry
        lax.fori_loop(0, GCH, ab, 0)
        pltpu.sync_copy(grow, h1pre_hbm.at[pl.ds(nbase + k * GCH, GCH)])


_sc1 = pl.kernel(
    _sc1_body,
    out_type=jax.ShapeDtypeStruct((N_PAD, HID), jnp.float32),
    mesh=_sc_mesh,
    scratch_types=[
        pltpu.VMEM((GCH,), jnp.int32),
        pltpu.VMEM((GCH, HID), jnp.float32),
        pltpu.VMEM((GCH, HID), jnp.float32),
        pltpu.SemaphoreType.DMA,
    ],
)


def _deg_body(d0_ref, d1_ref, o_ref):
    o_ref[...] = lax.rsqrt(d0_ref[:, :1] + d1_ref[:, :1] + 1.0)


def _deg_reduce(degp, br=1024):
    nblk = N_PAD // br
    return pl.pallas_call(
        _deg_body,
        grid=(nblk,),
        in_specs=[pl.BlockSpec((br, HID), lambda i: (i, 0)),
                  pl.BlockSpec((br, HID), lambda i: (i + nblk, 0))],
        out_specs=pl.BlockSpec((br, 1), lambda i: (i, 0)),
        out_shape=jax.ShapeDtypeStruct((N_PAD, 1), jnp.float32),
    )(degp, degp)


def _agg_body(hs_hbm, src_hbm, dst_hbm, accp_hbm,
              sidx, didx, rows, zbuf, acc, gsem):
    c = lax.axis_index("c")
    s = lax.axis_index("s")

    def zb(j, carry):
        for t in range(HID // 16):
            zbuf[j, pl.ds(t * 16, 16)] = jnp.zeros((16,), jnp.float32)
        return carry
    lax.fori_loop(0, ZROWS, zb, 0)

    def zc(r, carry):
        pltpu.sync_copy(zbuf, acc.at[pl.ds(s * RPS + r * ZROWS, ZROWS)])
        return carry
    lax.fori_loop(0, RPS // ZROWS, zc, 0)
    plsc.subcore_barrier()

    ebase = c * E_HALF + s * EPW

    def eb(i, carry):
        pltpu.sync_copy(src_hbm.at[pl.ds(ebase + i * CH, CH)], sidx)
        pltpu.async_copy(hs_hbm.at[sidx], rows, gsem).wait()
        pltpu.sync_copy(dst_hbm.at[pl.ds(ebase + i * CH, CH)], didx)
        pltpu.sync_copy(rows, acc.at[didx], add=True)
        return carry
    lax.fori_loop(0, CPW, eb, 0)

    plsc.subcore_barrier()
    pltpu.sync_copy(acc.at[pl.ds(s * RPS, RPS)],
                    accp_hbm.at[pl.ds(c * N_PAD + s * RPS, RPS)])


_sc_agg = pl.kernel(
    _agg_body,
    out_type=jax.ShapeDtypeStruct((2 * N_PAD, HID), jnp.float32),
    mesh=_sc_mesh,
    scratch_types=[
        pltpu.VMEM((CH,), jnp.int32),
        pltpu.VMEM((CH,), jnp.int32),
        pltpu.VMEM((CH, HID), jnp.float32),
        pltpu.VMEM((ZROWS, HID), jnp.float32),
        pltpu.VMEM_SHARED((N_PAD, HID), jnp.float32),
        pltpu.SemaphoreType.DMA,
    ],
)


def _hs_body(h_ref, d_ref, o_ref):
    o_ref[...] = h_ref[...] * d_ref[...]


def _hs_scale(h1pre, dinv_col, br=1024):
    nblk = N_PAD // br
    return pl.pallas_call(
        _hs_body,
        grid=(nblk,),
        in_specs=[pl.BlockSpec((br, HID), lambda i: (i, 0)),
                  pl.BlockSpec((br, 1), lambda i: (i, 0))],
        out_specs=pl.BlockSpec((br, HID), lambda i: (i, 0)),
        out_shape=jax.ShapeDtypeStruct((N_PAD, HID), jnp.float32),
    )(h1pre, dinv_col)


def _conv_body(a0_ref, a1_ref, hs_ref, d_ref, b_ref, w_ref, o_ref):
    dinv = d_ref[...]
    z = jnp.maximum(
        dinv * (a0_ref[...] + a1_ref[...] + hs_ref[...]) + b_ref[...], 0.0)
    o_ref[...] = jnp.dot(z, w_ref[...],
                         preferred_element_type=jnp.float32) * dinv


def _conv_mm(accp, hs, dinv_col, brow, w, br=1024):
    nblk = N_PAD // br
    return pl.pallas_call(
        _conv_body,
        grid=(nblk,),
        in_specs=[pl.BlockSpec((br, HID), lambda i: (i, 0)),
                  pl.BlockSpec((br, HID), lambda i: (i + nblk, 0)),
                  pl.BlockSpec((br, HID), lambda i: (i, 0)),
                  pl.BlockSpec((br, 1), lambda i: (i, 0)),
                  pl.BlockSpec((1, HID), lambda i: (0, 0)),
                  pl.BlockSpec((HID, HID), lambda i: (0, 0))],
        out_specs=pl.BlockSpec((br, HID), lambda i: (i, 0)),
        out_shape=jax.ShapeDtypeStruct((N_PAD, HID), jnp.float32),
    )(accp, accp, hs, dinv_col, brow, w)


def _final_body(acc_ref, hs_ref, dinv_ref, batch_ref, uv_ref, b2_ref,
                wm1_ref, bm1_ref, wm2_ref, bm2_ref, o_ref, z2_ref):
    dinv = dinv_ref[...]
    z2_ref[...] = jnp.maximum(
        dinv * (acc_ref[:N_PAD, :] + acc_ref[N_PAD:, :] + hs_ref[...])
        + b2_ref[...], 0.0)

    rowid = lax.broadcasted_iota(jnp.int32, (N_PAD, 1), 0)
    gsel = lax.broadcasted_iota(jnp.int32, (G, 1), 0)

    def gloop(g, feats):
        bcol = batch_ref[...]
        z2 = z2_ref[...]
        pool = jnp.max(jnp.where(bcol == g, z2, -jnp.inf), axis=0)
        ptr_g = jnp.sum((bcol < g).astype(jnp.int32))
        iu = jnp.minimum(ptr_g + uv_ref[2 * g], N - 1)
        iv = jnp.minimum(ptr_g + uv_ref[2 * g + 1], N - 1)
        hu = jnp.max(jnp.where(rowid == iu, z2, -jnp.inf), axis=0)
        hv = jnp.max(jnp.where(rowid == iv, z2, -jnp.inf), axis=0)
        row = jnp.concatenate(
            [hu, hv, jnp.abs(hu - hv), hu * hv, pool])[None, :]
        return jnp.where(gsel == g, row, feats)

    feats = lax.fori_loop(0, G, gloop,
                          jnp.zeros((G, 5 * HID), jnp.float32))

    hidden = jnp.maximum(
        jnp.dot(feats, wm1_ref[...],
                preferred_element_type=jnp.float32) + bm1_ref[...], 0.0)
    o_ref[...] = jnp.dot(hidden, wm2_ref[...],
                         preferred_element_type=jnp.float32) + bm2_ref[...]


def _final(accp, hs, dinv_col, batch_col, uv, b2r, wm1, bm1r, wm2, bm2r):
    return pl.pallas_call(
        _final_body,
        in_specs=[
            pl.BlockSpec((2 * N_PAD, HID), lambda: (0, 0)),
            pl.BlockSpec((N_PAD, HID), lambda: (0, 0)),
            pl.BlockSpec((N_PAD, 1), lambda: (0, 0)),
            pl.BlockSpec((N_PAD, 1), lambda: (0, 0)),
            pl.BlockSpec(memory_space=pltpu.SMEM),
            pl.BlockSpec((1, HID), lambda: (0, 0)),
            pl.BlockSpec((5 * HID, HID), lambda: (0, 0)),
            pl.BlockSpec((1, HID), lambda: (0, 0)),
            pl.BlockSpec((HID, 2), lambda: (0, 0)),
            pl.BlockSpec((1, 2), lambda: (0, 0)),
        ],
        out_specs=pl.BlockSpec((G, 2), lambda: (0, 0)),
        out_shape=jax.ShapeDtypeStruct((G, 2), jnp.float32),
        scratch_shapes=[pltpu.VMEM((N_PAD, HID), jnp.float32)],
    )(accp, hs, dinv_col, batch_col, uv, b2r, wm1, bm1r, wm2, bm2r)


def kernel(x, edge_index, drnl, batch, target_local, emb, W1, b1, W2, b2,
           Wm1, bm1, Wm2, bm2):
    src = edge_index[0].astype(jnp.int32)
    dst = edge_index[1].astype(jnp.int32)
    src_p = jnp.full((E_PAD,), PAD_NODE, jnp.int32).at[:E].set(src)
    dst_p = jnp.full((E_PAD,), PAD_NODE, jnp.int32).at[:E].set(dst)
    drnl_p = jnp.zeros((N_PAD,), jnp.int32).at[:N].set(drnl.astype(jnp.int32))
    batch_col = jnp.full((N_PAD, 1), G, jnp.int32).at[:N, 0].set(
        batch.astype(jnp.int32))
    x_p = jnp.zeros((N_PAD, D), jnp.float32).at[:N].set(x)
    uv = target_local.astype(jnp.int32)

    embw = _matmul(emb, W1[D:], emb.shape[0])
    xw = _matmul(x_p, W1[:D], 1024)
    degp = _sc_deg(dst_p)
    h1pre = _sc1(drnl_p, embw, xw)
    dinv_col = _deg_reduce(degp)
    hs1 = _hs_scale(h1pre, dinv_col)
    accp1 = _sc_agg(hs1, src_p, dst_p)
    hs2 = _conv_mm(accp1, hs1, dinv_col, b1.reshape(1, HID), W2)
    accp2 = _sc_agg(hs2, src_p, dst_p)
    return _final(accp2, hs2, dinv_col, batch_col, uv, b2.reshape(1, HID),
                  Wm1, bm1.reshape(1, HID), Wm2, bm2.reshape(1, 2))
